# baseline (speedup 1.0000x reference)
import jax
import jax.numpy as jnp
from jax import lax
from jax.experimental import pallas as pl
from jax.experimental.pallas import tpu as pltpu
from jax.experimental.pallas import tpu_sc as plsc

N = 10000
NP = 10240
E = 320000
EP = 327680
D = 64
D_IN = 128
NC = 2
NS = 16
L = 16
NW = NC * NS
CH = 128
NCHUNK = EP // (NW * CH)
GRP = CH // L
RPT = NP // NS
D2 = D + L


def _mlp_body(x_ref, w_ref, b_ref, h_ref, invn_ref):
    h = jnp.dot(x_ref[...], w_ref[...], preferred_element_type=jnp.float32)
    h = jnp.maximum(h + b_ref[...], 0.0)
    h_ref[...] = h
    s2 = jnp.sum(h * h, axis=1, keepdims=True)
    invn_ref[...] = 1.0 / jnp.maximum(jnp.sqrt(s2), 1e-12)


def _combine(acc_ref, den_ref, h_ref, invn_ref, beta_ref):
    h = h_ref[...]
    invn = invn_ref[...]
    s2 = jnp.sum(h * h, axis=1, keepdims=True)
    selfw = jnp.exp(beta_ref[...] * s2 * invn * invn)
    num = acc_ref[0] + acc_ref[1] + selfw * h
    den = den_ref[0, :, 0:1] + den_ref[1, :, 0:1] + selfw
    return num / den


def _combine_body(acc_ref, den_ref, h_ref, invn_ref, beta_ref,
                  h1_ref, invn1_ref):
    h1 = _combine(acc_ref, den_ref, h_ref, invn_ref, beta_ref)
    h1_ref[...] = h1
    s2 = jnp.sum(h1 * h1, axis=1, keepdims=True)
    invn1_ref[...] = 1.0 / jnp.maximum(jnp.sqrt(s2), 1e-12)


def _final_body(acc_ref, den_ref, h_ref, invn_ref, beta_ref, w_ref, b_ref,
                out_ref):
    h2 = _combine(acc_ref, den_ref, h_ref, invn_ref, beta_ref)
    z = jnp.dot(h2, w_ref[...], preferred_element_type=jnp.float32)
    z = z + b_ref[...]
    m = jnp.max(z, axis=1, keepdims=True)
    ez = jnp.exp(z - m)
    out_ref[...] = z - m - jnp.log(jnp.sum(ez, axis=1, keepdims=True))


def _prop_body(h_hbm, invn_hbm, src_hbm, dst_hbm, beta_hbm,
               acc_hbm, den_hbm,
               srcv, dstv, hs, hd, hs2, hd2, hw, denr, invnt, betav,
               accs, dens, sem1, sem2, sem3, sem4):
    cid = lax.axis_index("c")
    sid = lax.axis_index("s")
    wid = sid * NC + cid

    zero16 = jnp.zeros((L,), jnp.float32)

    def zfill_acc(i, c):
        for k in range(4):
            hw[i, pl.ds(k * L, L)] = zero16
        denr[i, pl.ds(0, L)] = zero16
        return c
    lax.fori_loop(0, CH, zfill_acc, 0)

    for k in range(5):
        pltpu.sync_copy(hw, accs.at[pl.ds(sid * RPT + k * CH, CH)])
        pltpu.sync_copy(denr, dens.at[pl.ds(sid * RPT + k * CH, CH)])

    pltpu.sync_copy(invn_hbm, invnt)
    pltpu.sync_copy(beta_hbm, betav)
    pltpu.sync_copy(src_hbm.at[pl.ds(wid * NCHUNK, NCHUNK)], srcv)
    pltpu.sync_copy(dst_hbm.at[pl.ds(wid * NCHUNK, NCHUNK)], dstv)

    plsc.subcore_barrier()

    def compute(c, hsx, hdx):
        @plsc.parallel_loop(0, GRP, 1, unroll=4)
        def group(g):
            sl = pl.ds(g * L, L)
            invns = plsc.load_gather(invnt, [srcv[c, sl]])
            invnd = plsc.load_gather(invnt, [dstv[c, sl]])
            sfac = invns * invnd * betav[pl.ds(0, L)]
            for j in range(L):
                e = g * L + j
                a0 = hsx[e, pl.ds(0, L)]
                a1 = hsx[e, pl.ds(L, L)]
                a2 = hsx[e, pl.ds(2 * L, L)]
                a3 = hsx[e, pl.ds(3 * L, L)]
                p = ((a0 * hdx[e, pl.ds(0, L)] + a1 * hdx[e, pl.ds(L, L)]) +
                     (a2 * hdx[e, pl.ds(2 * L, L)] + a3 * hdx[e, pl.ds(3 * L, L)]))
                wv = jnp.exp((zero16 + jnp.sum(p)) * sfac[j])
                denr[e, pl.ds(0, L)] = wv
                hw[e, pl.ds(0, L)] = a0 * wv
                hw[e, pl.ds(L, L)] = a1 * wv
                hw[e, pl.ds(2 * L, L)] = a2 * wv
                hw[e, pl.ds(3 * L, L)] = a3 * wv

        pltpu.sync_copy(hw, accs.at[dstv.at[c]], add=True)
        pltpu.sync_copy(denr, dens.at[dstv.at[c]], add=True)

    pltpu.async_copy(h_hbm.at[srcv.at[0]], hs, sem1)
    pltpu.async_copy(h_hbm.at[dstv.at[0]], hd, sem2)

    def pair(i, carry):
        c0 = 2 * i
        c1 = c0 + 1
        gb1 = pltpu.async_copy(h_hbm.at[srcv.at[c1]], hs2, sem3)
        gb2 = pltpu.async_copy(h_hbm.at[dstv.at[c1]], hd2, sem4)
        pltpu.make_async_copy(h_hbm.at[srcv.at[c0]], hs, sem1).wait()
        pltpu.make_async_copy(h_hbm.at[dstv.at[c0]], hd, sem2).wait()
        compute(c0, hs, hd)
        c0n = jnp.minimum(c0 + 2, NCHUNK - 1)
        pltpu.async_copy(h_hbm.at[srcv.at[c0n]], hs, sem1)
        pltpu.async_copy(h_hbm.at[dstv.at[c0n]], hd, sem2)
        gb1.wait()
        gb2.wait()
        compute(c1, hs2, hd2)
        return carry
    lax.fori_loop(0, NCHUNK // 2, pair, 0)

    pltpu.make_async_copy(h_hbm.at[srcv.at[NCHUNK - 1]], hs, sem1).wait()
    pltpu.make_async_copy(h_hbm.at[dstv.at[NCHUNK - 1]], hd, sem2).wait()

    plsc.subcore_barrier()

    r0 = sid * RPT
    for k in range(5):
        pltpu.sync_copy(accs.at[pl.ds(r0 + k * CH, CH)], hw)
        pltpu.sync_copy(hw, acc_hbm.at[cid, pl.ds(r0 + k * CH, CH)])
        pltpu.sync_copy(dens.at[pl.ds(r0 + k * CH, CH)], denr)
        pltpu.sync_copy(denr, den_hbm.at[cid, pl.ds(r0 + k * CH, CH)])


def _make_prop():
    mesh = plsc.VectorSubcoreMesh(core_axis_name="c", subcore_axis_name="s",
                                  num_cores=NC, num_subcores=NS)
    return pl.kernel(
        _prop_body,
        out_type=[
            jax.ShapeDtypeStruct((NC, NP, D), jnp.float32),
            jax.ShapeDtypeStruct((NC, NP, L), jnp.float32),
        ],
        mesh=mesh,
        compiler_params=pltpu.CompilerParams(needs_layout_passes=False, use_tc_tiling_on_sc=False),
        scratch_types=[
            pltpu.VMEM((NW * NCHUNK // NW, CH), jnp.int32),
            pltpu.VMEM((NCHUNK, CH), jnp.int32),
            pltpu.VMEM((CH, D), jnp.float32),
            pltpu.VMEM((CH, D), jnp.float32),
            pltpu.VMEM((CH, D), jnp.float32),
            pltpu.VMEM((CH, D), jnp.float32),
            pltpu.VMEM((CH, D), jnp.float32),
            pltpu.VMEM((CH, L), jnp.float32),
            pltpu.VMEM((NP,), jnp.float32),
            pltpu.VMEM((L,), jnp.float32),
            pltpu.VMEM_SHARED((NP, D), jnp.float32),
            pltpu.VMEM_SHARED((NP, L), jnp.float32),
            pltpu.SemaphoreType.DMA,
            pltpu.SemaphoreType.DMA,
            pltpu.SemaphoreType.DMA,
            pltpu.SemaphoreType.DMA,
        ],
    )


def kernel(x, edge_index, W1, b1, beta2, W2, b2):
    pad = jnp.full((2, EP - E), N, jnp.int32)
    ei = jnp.concatenate([edge_index.astype(jnp.int32), pad], axis=1)
    src = ei[0].reshape(EP // CH, CH)
    dst = ei[1].reshape(EP // CH, CH)
    xp = jnp.concatenate(
        [x, jnp.zeros((NP - N, D_IN), jnp.float32)], axis=0)

    h, invn = pl.pallas_call(
        _mlp_body,
        out_shape=[
            jax.ShapeDtypeStruct((NP, D), jnp.float32),
            jax.ShapeDtypeStruct((NP, 1), jnp.float32),
        ],
    )(xp, W1.T, b1.reshape(1, D))

    prop = _make_prop()
    combine = pl.pallas_call(
        _combine_body,
        out_shape=[
            jax.ShapeDtypeStruct((NP, D), jnp.float32),
            jax.ShapeDtypeStruct((NP, 1), jnp.float32),
        ],
    )
    final = pl.pallas_call(
        _final_body,
        out_shape=jax.ShapeDtypeStruct((NP, D), jnp.float32),
    )

    one16 = jnp.ones((L,), jnp.float32)
    one11 = jnp.ones((1, 1), jnp.float32)
    beta16 = jnp.broadcast_to(beta2.astype(jnp.float32), (L,))
    beta11 = beta2.astype(jnp.float32).reshape(1, 1)

    acc1, den1 = prop(h, invn.reshape(NP), src, dst, one16)
    h1, invn1 = combine(acc1, den1, h, invn, one11)
    acc2, den2 = prop(h1, invn1.reshape(NP), src, dst, beta16)
    out = final(acc2, den2, h1, invn1, beta11, W2.T, b2.reshape(1, D))
    return out[:N]


# speedup vs baseline: 1.2288x; 1.2288x over previous
import jax
import jax.numpy as jnp
from jax import lax
from jax.experimental import pallas as pl
from jax.experimental.pallas import tpu as pltpu
from jax.experimental.pallas import tpu_sc as plsc

N = 10000
NP = 10240
E = 320000
EP = 327680
D = 64
D_IN = 128
NC = 2
NS = 16
L = 16
NW = NC * NS
CH = 128
NCHUNK = EP // (NW * CH)
GRP = CH // L
RPT = NP // NS
D2 = D + L


def _mlp_body(x_ref, w_ref, b_ref, h_ref, invn_ref):
    h = jnp.dot(x_ref[...], w_ref[...], preferred_element_type=jnp.float32)
    h = jnp.maximum(h + b_ref[...], 0.0)
    h_ref[...] = h
    s2 = jnp.sum(h * h, axis=1, keepdims=True)
    invn_ref[...] = 1.0 / jnp.maximum(jnp.sqrt(s2), 1e-12)


def _combine(acc_ref, den_ref, h_ref, invn_ref, beta_ref):
    h = h_ref[...]
    invn = invn_ref[...]
    s2 = jnp.sum(h * h, axis=1, keepdims=True)
    selfw = jnp.exp(beta_ref[...] * s2 * invn * invn)
    num = acc_ref[0] + acc_ref[1] + selfw * h
    den = den_ref[0, :, 0:1] + den_ref[1, :, 0:1] + selfw
    return num / den


def _combine_body(acc_ref, den_ref, h_ref, invn_ref, beta_ref,
                  h1_ref, invn1_ref):
    h1 = _combine(acc_ref, den_ref, h_ref, invn_ref, beta_ref)
    h1_ref[...] = h1
    s2 = jnp.sum(h1 * h1, axis=1, keepdims=True)
    invn1_ref[...] = 1.0 / jnp.maximum(jnp.sqrt(s2), 1e-12)


def _final_body(acc_ref, den_ref, h_ref, invn_ref, beta_ref, w_ref, b_ref,
                out_ref):
    h2 = _combine(acc_ref, den_ref, h_ref, invn_ref, beta_ref)
    z = jnp.dot(h2, w_ref[...], preferred_element_type=jnp.float32)
    z = z + b_ref[...]
    m = jnp.max(z, axis=1, keepdims=True)
    ez = jnp.exp(z - m)
    out_ref[...] = z - m - jnp.log(jnp.sum(ez, axis=1, keepdims=True))


def _prop_body(h_hbm, invn_hbm, src_hbm, dst_hbm, beta_hbm,
               acc_hbm, den_hbm,
               srcv, dstv, hs, hd, hs2, hd2, hw, denr, invnt, betav,
               accs, dens, sem1, sem2, sem3, sem4):
    cid = lax.axis_index("c")
    sid = lax.axis_index("s")
    wid = sid * NC + cid

    zero16 = jnp.zeros((L,), jnp.float32)

    def zfill_acc(i, c):
        for k in range(4):
            hw[i, pl.ds(k * L, L)] = zero16
        denr[i, pl.ds(0, L)] = zero16
        return c
    lax.fori_loop(0, CH, zfill_acc, 0)

    for k in range(5):
        pltpu.sync_copy(hw, accs.at[pl.ds(sid * RPT + k * CH, CH)])
        pltpu.sync_copy(denr, dens.at[pl.ds(sid * RPT + k * CH, CH)])

    pltpu.sync_copy(invn_hbm, invnt)
    pltpu.sync_copy(beta_hbm, betav)
    pltpu.sync_copy(src_hbm.at[pl.ds(wid * NCHUNK, NCHUNK)], srcv)
    pltpu.sync_copy(dst_hbm.at[pl.ds(wid * NCHUNK, NCHUNK)], dstv)

    plsc.subcore_barrier()

    def compute(c, hsx, hdx):
        @plsc.parallel_loop(0, GRP, 1, unroll=1)
        def group(g):
            sl = pl.ds(g * L, L)
            invns = plsc.load_gather(invnt, [srcv[c, sl]])
            invnd = plsc.load_gather(invnt, [dstv[c, sl]])
            sfac = invns * invnd * betav[pl.ds(0, L)]
            for j in range(L):
                e = g * L + j
                a0 = hsx[e, pl.ds(0, L)]
                a1 = hsx[e, pl.ds(L, L)]
                a2 = hsx[e, pl.ds(2 * L, L)]
                a3 = hsx[e, pl.ds(3 * L, L)]
                p = ((a0 * hdx[e, pl.ds(0, L)] + a1 * hdx[e, pl.ds(L, L)]) +
                     (a2 * hdx[e, pl.ds(2 * L, L)] + a3 * hdx[e, pl.ds(3 * L, L)]))
                wv = jnp.exp((zero16 + jnp.sum(p)) * sfac[j])
                denr[e, pl.ds(0, L)] = wv
                hw[e, pl.ds(0, L)] = a0 * wv
                hw[e, pl.ds(L, L)] = a1 * wv
                hw[e, pl.ds(2 * L, L)] = a2 * wv
                hw[e, pl.ds(3 * L, L)] = a3 * wv

        pltpu.sync_copy(hw, accs.at[dstv.at[c]], add=True)
        pltpu.sync_copy(denr, dens.at[dstv.at[c]], add=True)

    pltpu.async_copy(h_hbm.at[srcv.at[0]], hs, sem1)
    pltpu.async_copy(h_hbm.at[dstv.at[0]], hd, sem2)

    def pair(i, carry):
        c0 = 2 * i
        c1 = c0 + 1
        gb1 = pltpu.async_copy(h_hbm.at[srcv.at[c1]], hs2, sem3)
        gb2 = pltpu.async_copy(h_hbm.at[dstv.at[c1]], hd2, sem4)
        pltpu.make_async_copy(h_hbm.at[srcv.at[c0]], hs, sem1).wait()
        pltpu.make_async_copy(h_hbm.at[dstv.at[c0]], hd, sem2).wait()
        compute(c0, hs, hd)
        c0n = jnp.minimum(c0 + 2, NCHUNK - 1)
        pltpu.async_copy(h_hbm.at[srcv.at[c0n]], hs, sem1)
        pltpu.async_copy(h_hbm.at[dstv.at[c0n]], hd, sem2)
        gb1.wait()
        gb2.wait()
        compute(c1, hs2, hd2)
        return carry
    lax.fori_loop(0, NCHUNK // 2, pair, 0)

    pltpu.make_async_copy(h_hbm.at[srcv.at[NCHUNK - 1]], hs, sem1).wait()
    pltpu.make_async_copy(h_hbm.at[dstv.at[NCHUNK - 1]], hd, sem2).wait()

    plsc.subcore_barrier()

    r0 = sid * RPT
    for k in range(5):
        pltpu.sync_copy(accs.at[pl.ds(r0 + k * CH, CH)], hw)
        pltpu.sync_copy(hw, acc_hbm.at[cid, pl.ds(r0 + k * CH, CH)])
        pltpu.sync_copy(dens.at[pl.ds(r0 + k * CH, CH)], denr)
        pltpu.sync_copy(denr, den_hbm.at[cid, pl.ds(r0 + k * CH, CH)])


def _make_prop():
    mesh = plsc.VectorSubcoreMesh(core_axis_name="c", subcore_axis_name="s",
                                  num_cores=NC, num_subcores=NS)
    return pl.kernel(
        _prop_body,
        out_type=[
            jax.ShapeDtypeStruct((NC, NP, D), jnp.float32),
            jax.ShapeDtypeStruct((NC, NP, L), jnp.float32),
        ],
        mesh=mesh,
        compiler_params=pltpu.CompilerParams(needs_layout_passes=False, use_tc_tiling_on_sc=False),
        scratch_types=[
            pltpu.VMEM((NW * NCHUNK // NW, CH), jnp.int32),
            pltpu.VMEM((NCHUNK, CH), jnp.int32),
            pltpu.VMEM((CH, D), jnp.float32),
            pltpu.VMEM((CH, D), jnp.float32),
            pltpu.VMEM((CH, D), jnp.float32),
            pltpu.VMEM((CH, D), jnp.float32),
            pltpu.VMEM((CH, D), jnp.float32),
            pltpu.VMEM((CH, L), jnp.float32),
            pltpu.VMEM((NP,), jnp.float32),
            pltpu.VMEM((L,), jnp.float32),
            pltpu.VMEM_SHARED((NP, D), jnp.float32),
            pltpu.VMEM_SHARED((NP, L), jnp.float32),
            pltpu.SemaphoreType.DMA,
            pltpu.SemaphoreType.DMA,
            pltpu.SemaphoreType.DMA,
            pltpu.SemaphoreType.DMA,
        ],
    )


def kernel(x, edge_index, W1, b1, beta2, W2, b2):
    pad = jnp.full((2, EP - E), N, jnp.int32)
    ei = jnp.concatenate([edge_index.astype(jnp.int32), pad], axis=1)
    src = ei[0].reshape(EP // CH, CH)
    dst = ei[1].reshape(EP // CH, CH)
    xp = jnp.concatenate(
        [x, jnp.zeros((NP - N, D_IN), jnp.float32)], axis=0)

    h, invn = pl.pallas_call(
        _mlp_body,
        out_shape=[
            jax.ShapeDtypeStruct((NP, D), jnp.float32),
            jax.ShapeDtypeStruct((NP, 1), jnp.float32),
        ],
    )(xp, W1.T, b1.reshape(1, D))

    prop = _make_prop()
    combine = pl.pallas_call(
        _combine_body,
        out_shape=[
            jax.ShapeDtypeStruct((NP, D), jnp.float32),
            jax.ShapeDtypeStruct((NP, 1), jnp.float32),
        ],
    )
    final = pl.pallas_call(
        _final_body,
        out_shape=jax.ShapeDtypeStruct((NP, D), jnp.float32),
    )

    one16 = jnp.ones((L,), jnp.float32)
    one11 = jnp.ones((1, 1), jnp.float32)
    beta16 = jnp.broadcast_to(beta2.astype(jnp.float32), (L,))
    beta11 = beta2.astype(jnp.float32).reshape(1, 1)

    acc1, den1 = prop(h, invn.reshape(NP), src, dst, one16)
    h1, invn1 = combine(acc1, den1, h, invn, one11)
    acc2, den2 = prop(h1, invn1.reshape(NP), src, dst, beta16)
    out = final(acc2, den2, h1, invn1, beta11, W2.T, b2.reshape(1, D))
    return out[:N]


# speedup vs baseline: 1.2370x; 1.0066x over previous
import jax
import jax.numpy as jnp
from jax import lax
from jax.experimental import pallas as pl
from jax.experimental.pallas import tpu as pltpu
from jax.experimental.pallas import tpu_sc as plsc

N = 10000
NP = 10240
E = 320000
EP = 327680
D = 64
D_IN = 128
NC = 2
NS = 16
L = 16
NW = NC * NS
CH = 128
NCHUNK = EP // (NW * CH)
GRP = CH // L
RPT = NP // NS
D2 = D + L


def _mlp_body(x_ref, w_ref, b_ref, h_ref, invn_ref):
    h = jnp.dot(x_ref[...], w_ref[...], preferred_element_type=jnp.float32)
    h = jnp.maximum(h + b_ref[...], 0.0)
    h_ref[...] = h
    s2 = jnp.sum(h * h, axis=1, keepdims=True)
    invn_ref[...] = 1.0 / jnp.maximum(jnp.sqrt(s2), 1e-12)


def _combine(acc_ref, den_ref, h_ref, invn_ref, beta_ref):
    h = h_ref[...]
    invn = invn_ref[...]
    s2 = jnp.sum(h * h, axis=1, keepdims=True)
    selfw = jnp.exp(beta_ref[...] * s2 * invn * invn)
    num = acc_ref[0] + acc_ref[1] + selfw * h
    den = den_ref[0, :, 0:1] + den_ref[1, :, 0:1] + selfw
    return num / den


def _combine_body(acc_ref, den_ref, h_ref, invn_ref, beta_ref,
                  h1_ref, invn1_ref):
    h1 = _combine(acc_ref, den_ref, h_ref, invn_ref, beta_ref)
    h1_ref[...] = h1
    s2 = jnp.sum(h1 * h1, axis=1, keepdims=True)
    invn1_ref[...] = 1.0 / jnp.maximum(jnp.sqrt(s2), 1e-12)


def _final_body(acc_ref, den_ref, h_ref, invn_ref, beta_ref, w_ref, b_ref,
                out_ref):
    h2 = _combine(acc_ref, den_ref, h_ref, invn_ref, beta_ref)
    z = jnp.dot(h2, w_ref[...], preferred_element_type=jnp.float32)
    z = z + b_ref[...]
    m = jnp.max(z, axis=1, keepdims=True)
    ez = jnp.exp(z - m)
    out_ref[...] = z - m - jnp.log(jnp.sum(ez, axis=1, keepdims=True))


def _prop_body(h_hbm, invn_hbm, src_hbm, dst_hbm, beta_hbm,
               acc_hbm, den_hbm,
               srcv, dstv, hs, hd, hs2, hd2, hw, denr, invnt, betav, sfact,
               accs, dens, sem1, sem2, sem3, sem4):
    cid = lax.axis_index("c")
    sid = lax.axis_index("s")
    wid = sid * NC + cid

    zero16 = jnp.zeros((L,), jnp.float32)

    def zfill_acc(i, c):
        for k in range(4):
            hw[i, pl.ds(k * L, L)] = zero16
        denr[i, pl.ds(0, L)] = zero16
        return c
    lax.fori_loop(0, CH, zfill_acc, 0)

    for k in range(5):
        pltpu.sync_copy(hw, accs.at[pl.ds(sid * RPT + k * CH, CH)])
        pltpu.sync_copy(denr, dens.at[pl.ds(sid * RPT + k * CH, CH)])

    pltpu.sync_copy(invn_hbm, invnt)
    pltpu.sync_copy(beta_hbm, betav)
    pltpu.sync_copy(src_hbm.at[pl.ds(wid * NCHUNK, NCHUNK)], srcv)
    pltpu.sync_copy(dst_hbm.at[pl.ds(wid * NCHUNK, NCHUNK)], dstv)

    plsc.subcore_barrier()

    def compute(c, hsx, hdx):
        @plsc.parallel_loop(0, GRP, 1, unroll=1)
        def group(g):
            sl = pl.ds(g * L, L)
            invns = plsc.load_gather(invnt, [srcv[c, sl]])
            invnd = plsc.load_gather(invnt, [dstv[c, sl]])
            sfact[sl] = invns * invnd * betav[pl.ds(0, L)]

        @plsc.parallel_loop(0, CH, 1, unroll=2)
        def edge(e):
            a0 = hsx[e, pl.ds(0, L)]
            a1 = hsx[e, pl.ds(L, L)]
            a2 = hsx[e, pl.ds(2 * L, L)]
            a3 = hsx[e, pl.ds(3 * L, L)]
            p = ((a0 * hdx[e, pl.ds(0, L)] + a1 * hdx[e, pl.ds(L, L)]) +
                 (a2 * hdx[e, pl.ds(2 * L, L)] + a3 * hdx[e, pl.ds(3 * L, L)]))
            sfacv = plsc.load_gather(sfact, [jnp.full((L,), e, jnp.int32)])
            wv = jnp.exp((zero16 + jnp.sum(p)) * sfacv)
            denr[e, pl.ds(0, L)] = wv
            hw[e, pl.ds(0, L)] = a0 * wv
            hw[e, pl.ds(L, L)] = a1 * wv
            hw[e, pl.ds(2 * L, L)] = a2 * wv
            hw[e, pl.ds(3 * L, L)] = a3 * wv

        pltpu.sync_copy(hw, accs.at[dstv.at[c]], add=True)
        pltpu.sync_copy(denr, dens.at[dstv.at[c]], add=True)

    pltpu.async_copy(h_hbm.at[srcv.at[0]], hs, sem1)
    pltpu.async_copy(h_hbm.at[dstv.at[0]], hd, sem2)

    def pair(i, carry):
        c0 = 2 * i
        c1 = c0 + 1
        gb1 = pltpu.async_copy(h_hbm.at[srcv.at[c1]], hs2, sem3)
        gb2 = pltpu.async_copy(h_hbm.at[dstv.at[c1]], hd2, sem4)
        pltpu.make_async_copy(h_hbm.at[srcv.at[c0]], hs, sem1).wait()
        pltpu.make_async_copy(h_hbm.at[dstv.at[c0]], hd, sem2).wait()
        compute(c0, hs, hd)
        c0n = jnp.minimum(c0 + 2, NCHUNK - 1)
        pltpu.async_copy(h_hbm.at[srcv.at[c0n]], hs, sem1)
        pltpu.async_copy(h_hbm.at[dstv.at[c0n]], hd, sem2)
        gb1.wait()
        gb2.wait()
        compute(c1, hs2, hd2)
        return carry
    lax.fori_loop(0, NCHUNK // 2, pair, 0)

    pltpu.make_async_copy(h_hbm.at[srcv.at[NCHUNK - 1]], hs, sem1).wait()
    pltpu.make_async_copy(h_hbm.at[dstv.at[NCHUNK - 1]], hd, sem2).wait()

    plsc.subcore_barrier()

    r0 = sid * RPT
    for k in range(5):
        pltpu.sync_copy(accs.at[pl.ds(r0 + k * CH, CH)], hw)
        pltpu.sync_copy(hw, acc_hbm.at[cid, pl.ds(r0 + k * CH, CH)])
        pltpu.sync_copy(dens.at[pl.ds(r0 + k * CH, CH)], denr)
        pltpu.sync_copy(denr, den_hbm.at[cid, pl.ds(r0 + k * CH, CH)])


def _make_prop():
    mesh = plsc.VectorSubcoreMesh(core_axis_name="c", subcore_axis_name="s",
                                  num_cores=NC, num_subcores=NS)
    return pl.kernel(
        _prop_body,
        out_type=[
            jax.ShapeDtypeStruct((NC, NP, D), jnp.float32),
            jax.ShapeDtypeStruct((NC, NP, L), jnp.float32),
        ],
        mesh=mesh,
        compiler_params=pltpu.CompilerParams(needs_layout_passes=False, use_tc_tiling_on_sc=False),
        scratch_types=[
            pltpu.VMEM((NW * NCHUNK // NW, CH), jnp.int32),
            pltpu.VMEM((NCHUNK, CH), jnp.int32),
            pltpu.VMEM((CH, D), jnp.float32),
            pltpu.VMEM((CH, D), jnp.float32),
            pltpu.VMEM((CH, D), jnp.float32),
            pltpu.VMEM((CH, D), jnp.float32),
            pltpu.VMEM((CH, D), jnp.float32),
            pltpu.VMEM((CH, L), jnp.float32),
            pltpu.VMEM((NP,), jnp.float32),
            pltpu.VMEM((L,), jnp.float32),
            pltpu.VMEM((CH,), jnp.float32),
            pltpu.VMEM_SHARED((NP, D), jnp.float32),
            pltpu.VMEM_SHARED((NP, L), jnp.float32),
            pltpu.SemaphoreType.DMA,
            pltpu.SemaphoreType.DMA,
            pltpu.SemaphoreType.DMA,
            pltpu.SemaphoreType.DMA,
        ],
    )


def kernel(x, edge_index, W1, b1, beta2, W2, b2):
    pad = jnp.full((2, EP - E), N, jnp.int32)
    ei = jnp.concatenate([edge_index.astype(jnp.int32), pad], axis=1)
    src = ei[0].reshape(EP // CH, CH)
    dst = ei[1].reshape(EP // CH, CH)
    xp = jnp.concatenate(
        [x, jnp.zeros((NP - N, D_IN), jnp.float32)], axis=0)

    h, invn = pl.pallas_call(
        _mlp_body,
        out_shape=[
            jax.ShapeDtypeStruct((NP, D), jnp.float32),
            jax.ShapeDtypeStruct((NP, 1), jnp.float32),
        ],
    )(xp, W1.T, b1.reshape(1, D))

    prop = _make_prop()
    combine = pl.pallas_call(
        _combine_body,
        out_shape=[
            jax.ShapeDtypeStruct((NP, D), jnp.float32),
            jax.ShapeDtypeStruct((NP, 1), jnp.float32),
        ],
    )
    final = pl.pallas_call(
        _final_body,
        out_shape=jax.ShapeDtypeStruct((NP, D), jnp.float32),
    )

    one16 = jnp.ones((L,), jnp.float32)
    one11 = jnp.ones((1, 1), jnp.float32)
    beta16 = jnp.broadcast_to(beta2.astype(jnp.float32), (L,))
    beta11 = beta2.astype(jnp.float32).reshape(1, 1)

    acc1, den1 = prop(h, invn.reshape(NP), src, dst, one16)
    h1, invn1 = combine(acc1, den1, h, invn, one11)
    acc2, den2 = prop(h1, invn1.reshape(NP), src, dst, beta16)
    out = final(acc2, den2, h1, invn1, beta11, W2.T, b2.reshape(1, D))
    return out[:N]


# speedup vs baseline: 1.2371x; 1.0001x over previous
import jax
import jax.numpy as jnp
from jax import lax
from jax.experimental import pallas as pl
from jax.experimental.pallas import tpu as pltpu
from jax.experimental.pallas import tpu_sc as plsc

N = 10000
NP = 10240
E = 320000
EP = 327680
D = 64
D_IN = 128
NC = 2
NS = 16
L = 16
NW = NC * NS
CH = 128
NCHUNK = EP // (NW * CH)
GRP = CH // L
RPT = NP // NS
D2 = D + L


def _mlp_body(x_ref, w_ref, b_ref, h_ref, invn_ref):
    h = jnp.dot(x_ref[...], w_ref[...], preferred_element_type=jnp.float32)
    h = jnp.maximum(h + b_ref[...], 0.0)
    h_ref[...] = h
    s2 = jnp.sum(h * h, axis=1, keepdims=True)
    invn_ref[...] = 1.0 / jnp.maximum(jnp.sqrt(s2), 1e-12)


def _combine(acc_ref, den_ref, h_ref, invn_ref, beta_ref):
    h = h_ref[...]
    invn = invn_ref[...]
    s2 = jnp.sum(h * h, axis=1, keepdims=True)
    selfw = jnp.exp(beta_ref[...] * s2 * invn * invn)
    num = acc_ref[0] + acc_ref[1] + selfw * h
    den = den_ref[0, :, 0:1] + den_ref[1, :, 0:1] + selfw
    return num / den


def _combine_body(acc_ref, den_ref, h_ref, invn_ref, beta_ref,
                  h1_ref, invn1_ref):
    h1 = _combine(acc_ref, den_ref, h_ref, invn_ref, beta_ref)
    h1_ref[...] = h1
    s2 = jnp.sum(h1 * h1, axis=1, keepdims=True)
    invn1_ref[...] = 1.0 / jnp.maximum(jnp.sqrt(s2), 1e-12)


def _final_body(acc_ref, den_ref, h_ref, invn_ref, beta_ref, w_ref, b_ref,
                out_ref):
    h2 = _combine(acc_ref, den_ref, h_ref, invn_ref, beta_ref)
    z = jnp.dot(h2, w_ref[...], preferred_element_type=jnp.float32)
    z = z + b_ref[...]
    m = jnp.max(z, axis=1, keepdims=True)
    ez = jnp.exp(z - m)
    out_ref[...] = z - m - jnp.log(jnp.sum(ez, axis=1, keepdims=True))


def _prop_body(h_hbm, invn_hbm, src_hbm, dst_hbm, beta_hbm,
               acc_hbm, den_hbm,
               srcv, dstv, hs, hd, hs2, hd2, hw, denr, invnt, betav, sfact,
               accs, dens, sem1, sem2, sem3, sem4):
    cid = lax.axis_index("c")
    sid = lax.axis_index("s")
    wid = sid * NC + cid

    zero16 = jnp.zeros((L,), jnp.float32)

    def zfill_acc(i, c):
        for k in range(4):
            hw[i, pl.ds(k * L, L)] = zero16
        denr[i, pl.ds(0, L)] = zero16
        return c
    lax.fori_loop(0, CH, zfill_acc, 0)

    for k in range(5):
        pltpu.sync_copy(hw, accs.at[pl.ds(sid * RPT + k * CH, CH)])
        pltpu.sync_copy(denr, dens.at[pl.ds(sid * RPT + k * CH, CH)])

    pltpu.sync_copy(invn_hbm, invnt)
    pltpu.sync_copy(beta_hbm, betav)
    pltpu.sync_copy(src_hbm.at[pl.ds(wid * NCHUNK, NCHUNK)], srcv)
    pltpu.sync_copy(dst_hbm.at[pl.ds(wid * NCHUNK, NCHUNK)], dstv)

    plsc.subcore_barrier()

    def compute(c, hsx, hdx):
        @plsc.parallel_loop(0, GRP, 1, unroll=1)
        def group(g):
            sl = pl.ds(g * L, L)
            invns = plsc.load_gather(invnt, [srcv[c, sl]])
            invnd = plsc.load_gather(invnt, [dstv[c, sl]])
            sfact[sl] = invns * invnd * betav[pl.ds(0, L)]

        @plsc.parallel_loop(0, CH, 1, unroll=4)
        def edge(e):
            a0 = hsx[e, pl.ds(0, L)]
            a1 = hsx[e, pl.ds(L, L)]
            a2 = hsx[e, pl.ds(2 * L, L)]
            a3 = hsx[e, pl.ds(3 * L, L)]
            p = ((a0 * hdx[e, pl.ds(0, L)] + a1 * hdx[e, pl.ds(L, L)]) +
                 (a2 * hdx[e, pl.ds(2 * L, L)] + a3 * hdx[e, pl.ds(3 * L, L)]))
            sfacv = plsc.load_gather(sfact, [jnp.full((L,), e, jnp.int32)])
            wv = jnp.exp((zero16 + jnp.sum(p)) * sfacv)
            denr[e, pl.ds(0, L)] = wv
            hw[e, pl.ds(0, L)] = a0 * wv
            hw[e, pl.ds(L, L)] = a1 * wv
            hw[e, pl.ds(2 * L, L)] = a2 * wv
            hw[e, pl.ds(3 * L, L)] = a3 * wv

        pltpu.sync_copy(hw, accs.at[dstv.at[c]], add=True)
        pltpu.sync_copy(denr, dens.at[dstv.at[c]], add=True)

    pltpu.async_copy(h_hbm.at[srcv.at[0]], hs, sem1)
    pltpu.async_copy(h_hbm.at[dstv.at[0]], hd, sem2)

    def pair(i, carry):
        c0 = 2 * i
        c1 = c0 + 1
        gb1 = pltpu.async_copy(h_hbm.at[srcv.at[c1]], hs2, sem3)
        gb2 = pltpu.async_copy(h_hbm.at[dstv.at[c1]], hd2, sem4)
        pltpu.make_async_copy(h_hbm.at[srcv.at[c0]], hs, sem1).wait()
        pltpu.make_async_copy(h_hbm.at[dstv.at[c0]], hd, sem2).wait()
        compute(c0, hs, hd)
        c0n = jnp.minimum(c0 + 2, NCHUNK - 1)
        pltpu.async_copy(h_hbm.at[srcv.at[c0n]], hs, sem1)
        pltpu.async_copy(h_hbm.at[dstv.at[c0n]], hd, sem2)
        gb1.wait()
        gb2.wait()
        compute(c1, hs2, hd2)
        return carry
    lax.fori_loop(0, NCHUNK // 2, pair, 0)

    pltpu.make_async_copy(h_hbm.at[srcv.at[NCHUNK - 1]], hs, sem1).wait()
    pltpu.make_async_copy(h_hbm.at[dstv.at[NCHUNK - 1]], hd, sem2).wait()

    plsc.subcore_barrier()

    r0 = sid * RPT
    for k in range(5):
        pltpu.sync_copy(accs.at[pl.ds(r0 + k * CH, CH)], hw)
        pltpu.sync_copy(hw, acc_hbm.at[cid, pl.ds(r0 + k * CH, CH)])
        pltpu.sync_copy(dens.at[pl.ds(r0 + k * CH, CH)], denr)
        pltpu.sync_copy(denr, den_hbm.at[cid, pl.ds(r0 + k * CH, CH)])


def _make_prop():
    mesh = plsc.VectorSubcoreMesh(core_axis_name="c", subcore_axis_name="s",
                                  num_cores=NC, num_subcores=NS)
    return pl.kernel(
        _prop_body,
        out_type=[
            jax.ShapeDtypeStruct((NC, NP, D), jnp.float32),
            jax.ShapeDtypeStruct((NC, NP, L), jnp.float32),
        ],
        mesh=mesh,
        compiler_params=pltpu.CompilerParams(needs_layout_passes=False, use_tc_tiling_on_sc=False),
        scratch_types=[
            pltpu.VMEM((NW * NCHUNK // NW, CH), jnp.int32),
            pltpu.VMEM((NCHUNK, CH), jnp.int32),
            pltpu.VMEM((CH, D), jnp.float32),
            pltpu.VMEM((CH, D), jnp.float32),
            pltpu.VMEM((CH, D), jnp.float32),
            pltpu.VMEM((CH, D), jnp.float32),
            pltpu.VMEM((CH, D), jnp.float32),
            pltpu.VMEM((CH, L), jnp.float32),
            pltpu.VMEM((NP,), jnp.float32),
            pltpu.VMEM((L,), jnp.float32),
            pltpu.VMEM((CH,), jnp.float32),
            pltpu.VMEM_SHARED((NP, D), jnp.float32),
            pltpu.VMEM_SHARED((NP, L), jnp.float32),
            pltpu.SemaphoreType.DMA,
            pltpu.SemaphoreType.DMA,
            pltpu.SemaphoreType.DMA,
            pltpu.SemaphoreType.DMA,
        ],
    )


def kernel(x, edge_index, W1, b1, beta2, W2, b2):
    pad = jnp.full((2, EP - E), N, jnp.int32)
    ei = jnp.concatenate([edge_index.astype(jnp.int32), pad], axis=1)
    src = ei[0].reshape(EP // CH, CH)
    dst = ei[1].reshape(EP // CH, CH)
    xp = jnp.concatenate(
        [x, jnp.zeros((NP - N, D_IN), jnp.float32)], axis=0)

    h, invn = pl.pallas_call(
        _mlp_body,
        out_shape=[
            jax.ShapeDtypeStruct((NP, D), jnp.float32),
            jax.ShapeDtypeStruct((NP, 1), jnp.float32),
        ],
    )(xp, W1.T, b1.reshape(1, D))

    prop = _make_prop()
    combine = pl.pallas_call(
        _combine_body,
        out_shape=[
            jax.ShapeDtypeStruct((NP, D), jnp.float32),
            jax.ShapeDtypeStruct((NP, 1), jnp.float32),
        ],
    )
    final = pl.pallas_call(
        _final_body,
        out_shape=jax.ShapeDtypeStruct((NP, D), jnp.float32),
    )

    one16 = jnp.ones((L,), jnp.float32)
    one11 = jnp.ones((1, 1), jnp.float32)
    beta16 = jnp.broadcast_to(beta2.astype(jnp.float32), (L,))
    beta11 = beta2.astype(jnp.float32).reshape(1, 1)

    acc1, den1 = prop(h, invn.reshape(NP), src, dst, one16)
    h1, invn1 = combine(acc1, den1, h, invn, one11)
    acc2, den2 = prop(h1, invn1.reshape(NP), src, dst, beta16)
    out = final(acc2, den2, h1, invn1, beta11, W2.T, b2.reshape(1, D))
    return out[:N]


# speedup vs baseline: 1.2445x; 1.0060x over previous
import jax
import jax.numpy as jnp
from jax import lax
from jax.experimental import pallas as pl
from jax.experimental.pallas import tpu as pltpu
from jax.experimental.pallas import tpu_sc as plsc

N = 10000
NP = 10240
E = 320000
EP = 327680
D = 64
D_IN = 128
NC = 2
NS = 16
L = 16
NW = NC * NS
CH = 64
NCHUNK = EP // (NW * CH)
GRP = CH // L
RPT = NP // NS
D2 = D + L


def _mlp_body(x_ref, w_ref, b_ref, h_ref, invn_ref):
    h = jnp.dot(x_ref[...], w_ref[...], preferred_element_type=jnp.float32)
    h = jnp.maximum(h + b_ref[...], 0.0)
    h_ref[...] = h
    s2 = jnp.sum(h * h, axis=1, keepdims=True)
    invn_ref[...] = 1.0 / jnp.maximum(jnp.sqrt(s2), 1e-12)


def _combine(acc_ref, den_ref, h_ref, invn_ref, beta_ref):
    h = h_ref[...]
    invn = invn_ref[...]
    s2 = jnp.sum(h * h, axis=1, keepdims=True)
    selfw = jnp.exp(beta_ref[...] * s2 * invn * invn)
    num = acc_ref[0] + acc_ref[1] + selfw * h
    den = den_ref[0, :, 0:1] + den_ref[1, :, 0:1] + selfw
    return num / den


def _combine_body(acc_ref, den_ref, h_ref, invn_ref, beta_ref,
                  h1_ref, invn1_ref):
    h1 = _combine(acc_ref, den_ref, h_ref, invn_ref, beta_ref)
    h1_ref[...] = h1
    s2 = jnp.sum(h1 * h1, axis=1, keepdims=True)
    invn1_ref[...] = 1.0 / jnp.maximum(jnp.sqrt(s2), 1e-12)


def _final_body(acc_ref, den_ref, h_ref, invn_ref, beta_ref, w_ref, b_ref,
                out_ref):
    h2 = _combine(acc_ref, den_ref, h_ref, invn_ref, beta_ref)
    z = jnp.dot(h2, w_ref[...], preferred_element_type=jnp.float32)
    z = z + b_ref[...]
    m = jnp.max(z, axis=1, keepdims=True)
    ez = jnp.exp(z - m)
    out_ref[...] = z - m - jnp.log(jnp.sum(ez, axis=1, keepdims=True))


def _prop_body(h_hbm, invn_hbm, src_hbm, dst_hbm, beta_hbm,
               acc_hbm, den_hbm,
               srcv, dstv, hs, hd, hs2, hd2, hw, denr, hw2, denr2,
               invnt, betav, sfact,
               accs, dens, sem1, sem2, sem3, sem4,
               sema1, sema2, semb1, semb2):
    cid = lax.axis_index("c")
    sid = lax.axis_index("s")
    wid = sid * NC + cid

    zero16 = jnp.zeros((L,), jnp.float32)

    def zfill_acc(i, c):
        for k in range(4):
            hw[i, pl.ds(k * L, L)] = zero16
        denr[i, pl.ds(0, L)] = zero16
        return c
    lax.fori_loop(0, CH, zfill_acc, 0)

    for k in range(RPT // CH):
        pltpu.sync_copy(hw, accs.at[pl.ds(sid * RPT + k * CH, CH)])
        pltpu.sync_copy(denr, dens.at[pl.ds(sid * RPT + k * CH, CH)])

    pltpu.sync_copy(invn_hbm, invnt)
    pltpu.sync_copy(beta_hbm, betav)
    pltpu.sync_copy(src_hbm.at[pl.ds(wid * NCHUNK, NCHUNK)], srcv)
    pltpu.sync_copy(dst_hbm.at[pl.ds(wid * NCHUNK, NCHUNK)], dstv)

    plsc.subcore_barrier()

    def compute(c, hsx, hdx, hwx, denrx, semacc, semden, first=False):
        @plsc.parallel_loop(0, GRP, 1, unroll=1)
        def group(g):
            sl = pl.ds(g * L, L)
            invns = plsc.load_gather(invnt, [srcv[c, sl]])
            invnd = plsc.load_gather(invnt, [dstv[c, sl]])
            sfact[sl] = invns * invnd * betav[pl.ds(0, L)]

        if not first:
            pltpu.make_async_copy(hwx, accs.at[dstv.at[c]], semacc).wait()
            pltpu.make_async_copy(denrx, dens.at[dstv.at[c]], semden).wait()

        @plsc.parallel_loop(0, CH, 1, unroll=4)
        def edge(e):
            a0 = hsx[e, pl.ds(0, L)]
            a1 = hsx[e, pl.ds(L, L)]
            a2 = hsx[e, pl.ds(2 * L, L)]
            a3 = hsx[e, pl.ds(3 * L, L)]
            p = ((a0 * hdx[e, pl.ds(0, L)] + a1 * hdx[e, pl.ds(L, L)]) +
                 (a2 * hdx[e, pl.ds(2 * L, L)] + a3 * hdx[e, pl.ds(3 * L, L)]))
            sfacv = plsc.load_gather(sfact, [jnp.full((L,), e, jnp.int32)])
            wv = jnp.exp((zero16 + jnp.sum(p)) * sfacv)
            denrx[e, pl.ds(0, L)] = wv
            hwx[e, pl.ds(0, L)] = a0 * wv
            hwx[e, pl.ds(L, L)] = a1 * wv
            hwx[e, pl.ds(2 * L, L)] = a2 * wv
            hwx[e, pl.ds(3 * L, L)] = a3 * wv

        pltpu.async_copy(hwx, accs.at[dstv.at[c]], semacc, add=True)
        pltpu.async_copy(denrx, dens.at[dstv.at[c]], semden, add=True)

    pltpu.async_copy(h_hbm.at[srcv.at[0]], hs, sem1)
    pltpu.async_copy(h_hbm.at[dstv.at[0]], hd, sem2)

    def pair_body(i, first):
        c0 = 2 * i
        c1 = c0 + 1
        gb1 = pltpu.async_copy(h_hbm.at[srcv.at[c1]], hs2, sem3)
        gb2 = pltpu.async_copy(h_hbm.at[dstv.at[c1]], hd2, sem4)
        pltpu.make_async_copy(h_hbm.at[srcv.at[c0]], hs, sem1).wait()
        pltpu.make_async_copy(h_hbm.at[dstv.at[c0]], hd, sem2).wait()
        compute(c0, hs, hd, hw, denr, sema1, sema2, first=first)
        c0n = jnp.minimum(c0 + 2, NCHUNK - 1)
        pltpu.async_copy(h_hbm.at[srcv.at[c0n]], hs, sem1)
        pltpu.async_copy(h_hbm.at[dstv.at[c0n]], hd, sem2)
        gb1.wait()
        gb2.wait()
        compute(c1, hs2, hd2, hw2, denr2, semb1, semb2, first=first)

    pair_body(0, True)

    def pair(i, carry):
        pair_body(i, False)
        return carry
    lax.fori_loop(1, NCHUNK // 2, pair, 0)

    pltpu.make_async_copy(h_hbm.at[srcv.at[NCHUNK - 1]], hs, sem1).wait()
    pltpu.make_async_copy(h_hbm.at[dstv.at[NCHUNK - 1]], hd, sem2).wait()
    lastc = NCHUNK - 1
    pltpu.make_async_copy(hw, accs.at[dstv.at[lastc]], sema1).wait()
    pltpu.make_async_copy(denr, dens.at[dstv.at[lastc]], sema2).wait()
    pltpu.make_async_copy(hw2, accs.at[dstv.at[lastc]], semb1).wait()
    pltpu.make_async_copy(denr2, dens.at[dstv.at[lastc]], semb2).wait()

    plsc.subcore_barrier()

    r0 = sid * RPT
    for k in range(RPT // CH):
        pltpu.sync_copy(accs.at[pl.ds(r0 + k * CH, CH)], hw)
        pltpu.sync_copy(hw, acc_hbm.at[cid, pl.ds(r0 + k * CH, CH)])
        pltpu.sync_copy(dens.at[pl.ds(r0 + k * CH, CH)], denr)
        pltpu.sync_copy(denr, den_hbm.at[cid, pl.ds(r0 + k * CH, CH)])


def _make_prop():
    mesh = plsc.VectorSubcoreMesh(core_axis_name="c", subcore_axis_name="s",
                                  num_cores=NC, num_subcores=NS)
    return pl.kernel(
        _prop_body,
        out_type=[
            jax.ShapeDtypeStruct((NC, NP, D), jnp.float32),
            jax.ShapeDtypeStruct((NC, NP, L), jnp.float32),
        ],
        mesh=mesh,
        compiler_params=pltpu.CompilerParams(needs_layout_passes=False, use_tc_tiling_on_sc=False),
        scratch_types=[
            pltpu.VMEM((NW * NCHUNK // NW, CH), jnp.int32),
            pltpu.VMEM((NCHUNK, CH), jnp.int32),
            pltpu.VMEM((CH, D), jnp.float32),
            pltpu.VMEM((CH, D), jnp.float32),
            pltpu.VMEM((CH, D), jnp.float32),
            pltpu.VMEM((CH, D), jnp.float32),
            pltpu.VMEM((CH, D), jnp.float32),
            pltpu.VMEM((CH, L), jnp.float32),
            pltpu.VMEM((CH, D), jnp.float32),
            pltpu.VMEM((CH, L), jnp.float32),
            pltpu.VMEM((NP,), jnp.float32),
            pltpu.VMEM((L,), jnp.float32),
            pltpu.VMEM((CH,), jnp.float32),
            pltpu.VMEM_SHARED((NP, D), jnp.float32),
            pltpu.VMEM_SHARED((NP, L), jnp.float32),
            pltpu.SemaphoreType.DMA,
            pltpu.SemaphoreType.DMA,
            pltpu.SemaphoreType.DMA,
            pltpu.SemaphoreType.DMA,
            pltpu.SemaphoreType.DMA,
            pltpu.SemaphoreType.DMA,
            pltpu.SemaphoreType.DMA,
            pltpu.SemaphoreType.DMA,
        ],
    )


def kernel(x, edge_index, W1, b1, beta2, W2, b2):
    pad = jnp.full((2, EP - E), N, jnp.int32)
    ei = jnp.concatenate([edge_index.astype(jnp.int32), pad], axis=1)
    src = ei[0].reshape(EP // CH, CH)
    dst = ei[1].reshape(EP // CH, CH)
    xp = jnp.concatenate(
        [x, jnp.zeros((NP - N, D_IN), jnp.float32)], axis=0)

    h, invn = pl.pallas_call(
        _mlp_body,
        out_shape=[
            jax.ShapeDtypeStruct((NP, D), jnp.float32),
            jax.ShapeDtypeStruct((NP, 1), jnp.float32),
        ],
    )(xp, W1.T, b1.reshape(1, D))

    prop = _make_prop()
    combine = pl.pallas_call(
        _combine_body,
        out_shape=[
            jax.ShapeDtypeStruct((NP, D), jnp.float32),
            jax.ShapeDtypeStruct((NP, 1), jnp.float32),
        ],
    )
    final = pl.pallas_call(
        _final_body,
        out_shape=jax.ShapeDtypeStruct((NP, D), jnp.float32),
    )

    one16 = jnp.ones((L,), jnp.float32)
    one11 = jnp.ones((1, 1), jnp.float32)
    beta16 = jnp.broadcast_to(beta2.astype(jnp.float32), (L,))
    beta11 = beta2.astype(jnp.float32).reshape(1, 1)

    acc1, den1 = prop(h, invn.reshape(NP), src, dst, one16)
    h1, invn1 = combine(acc1, den1, h, invn, one11)
    acc2, den2 = prop(h1, invn1.reshape(NP), src, dst, beta16)
    out = final(acc2, den2, h1, invn1, beta11, W2.T, b2.reshape(1, D))
    return out[:N]
